# baseline (speedup 1.0000x reference)
import jax
import jax.numpy as jnp
from jax import lax
from jax.experimental import pallas as pl


def _body(x_ref, p_ref, v_ref):
    step = x_ref[:, 0, 0:1]
    v_ref[:, :] = 100.0 - step
    par = step - 2.0 * jnp.floor(step * 0.5)
    any_odd = jnp.max(par) > 0.5
    any_even = jnp.min(par) < 0.5
    c0 = jnp.where(any_even, 0.8, 0.2)
    c1 = jnp.where(any_odd, 0.8, 0.2)
    col = lax.broadcasted_iota(jnp.int32, (p_ref.shape[0], 2), 1)
    p_ref[:, :] = jnp.where(col == 0, c0, c1)


def kernel(x):
    B = x.shape[0]
    W = x.shape[3]
    P, v = pl.pallas_call(
        _body,
        grid=(1,),
        in_specs=[pl.BlockSpec((B, None, 8, W), lambda i: (0, 0, 0, 0))],
        out_specs=(
            pl.BlockSpec((B, 2), lambda i: (0, 0)),
            pl.BlockSpec((B, 1), lambda i: (0, 0)),
        ),
        out_shape=(
            jax.ShapeDtypeStruct((B, 2), jnp.float32),
            jax.ShapeDtypeStruct((B, 1), jnp.float32),
        ),
    )(x)
    return (P, v)


# speedup vs baseline: 1.0194x; 1.0194x over previous
import jax
import jax.numpy as jnp
from jax import lax
from jax.experimental import pallas as pl
from jax.experimental.pallas import tpu as pltpu

NSTREAM = 16


def _body(x_hbm, p_ref, v_ref, face, sems):
    B = v_ref.shape[0]
    chunk = B // NSTREAM
    for k in range(NSTREAM):
        pltpu.make_async_copy(
            x_hbm.at[pl.ds(k * chunk, chunk), 0, 0],
            face.at[pl.ds(k * chunk, chunk)],
            sems.at[k],
        ).start()
    for k in range(NSTREAM):
        pltpu.make_async_copy(
            x_hbm.at[pl.ds(k * chunk, chunk), 0, 0],
            face.at[pl.ds(k * chunk, chunk)],
            sems.at[k],
        ).wait()
    step = face[:, 0:1]
    v_ref[:, :] = 100.0 - step
    par = jnp.bitwise_and(step.astype(jnp.int32), 1)
    any_odd = jnp.max(par) > 0
    any_even = jnp.min(par) < 1
    c0 = jnp.where(any_even, 0.8, 0.2)
    c1 = jnp.where(any_odd, 0.8, 0.2)
    col = lax.broadcasted_iota(jnp.int32, (B, 2), 1)
    p_ref[:, :] = jnp.where(col == 0, c0, c1)


def kernel(x):
    B = x.shape[0]
    W = x.shape[3]
    P, v = pl.pallas_call(
        _body,
        in_specs=[pl.BlockSpec(memory_space=pl.ANY)],
        out_specs=(
            pl.BlockSpec((B, 2), lambda: (0, 0)),
            pl.BlockSpec((B, 1), lambda: (0, 0)),
        ),
        out_shape=(
            jax.ShapeDtypeStruct((B, 2), jnp.float32),
            jax.ShapeDtypeStruct((B, 1), jnp.float32),
        ),
        scratch_shapes=[
            pltpu.VMEM((B, W), jnp.float32),
            pltpu.SemaphoreType.DMA((NSTREAM,)),
        ],
    )(x)
    return (P, v)
